# baseline (speedup 1.0000x reference)
import jax
import jax.numpy as jnp
from jax import lax
from jax.experimental import pallas as pl
from jax.experimental.pallas import tpu as pltpu
from jax.experimental.pallas import tpu_sc as plsc

_DIM = 1024
_NC = 2
_NS = 16
_NW = _NC * _NS
_BATCH = 16


def _body(x_hbm, table_hbm, out_hbm, idx_v, sem):
    b_per_w = x_hbm.shape[0] // _NW
    wid = lax.axis_index("s") * _NC + lax.axis_index("c")
    base = wid * b_per_w
    pltpu.sync_copy(x_hbm.at[pl.ds(base, b_per_w)], idx_v)

    def drain(count):
        @pl.loop(0, count)
        def _(_):
            pltpu.make_async_copy(
                table_hbm.at[pl.ds(0, 1)], out_hbm.at[pl.ds(0, 1)], sem
            ).wait()

    @pl.loop(0, b_per_w // _BATCH)
    def _(g):
        vv = idx_v[pl.ds(g * _BATCH, _BATCH)]
        for e in range(_BATCH):
            pltpu.async_copy(
                table_hbm.at[pl.ds(vv[e], 1)],
                out_hbm.at[pl.ds(base + g * _BATCH + e, 1)],
                sem,
            )
        drain(_BATCH)


def kernel(x, table):
    n = x.shape[0]
    mesh = plsc.VectorSubcoreMesh(
        core_axis_name="c", subcore_axis_name="s",
        num_cores=_NC, num_subcores=_NS,
    )
    f = pl.kernel(
        _body,
        out_type=jax.ShapeDtypeStruct((n, _DIM), jnp.float32),
        mesh=mesh,
        scratch_types=[
            pltpu.VMEM((n // _NW,), jnp.int32),
            pltpu.SemaphoreType.DMA,
        ],
    )
    return f(x.astype(jnp.int32), table)


# speedup vs baseline: 36.4840x; 36.4840x over previous
import jax
import jax.numpy as jnp
from jax import lax
from jax.experimental import pallas as pl
from jax.experimental.pallas import tpu as pltpu
from jax.experimental.pallas import tpu_sc as plsc

_DIM = 1024
_NC = 2
_NS = 16
_NW = _NC * _NS
_CHUNK = 32


def _body(x_hbm, table_hbm, out_hbm, idx_v, rows_a, rows_b,
          sin_a, sin_b, sout_a, sout_b):
    b_per_w = x_hbm.shape[0] // _NW
    nsteps = b_per_w // _CHUNK
    wid = lax.axis_index("s") * _NC + lax.axis_index("c")
    base = wid * b_per_w
    bufs = (rows_a, rows_b)
    sin = (sin_a, sin_b)
    sout = (sout_a, sout_b)

    pltpu.sync_copy(x_hbm.at[pl.ds(base, b_per_w)], idx_v)

    def in_start(j, b):
        pltpu.async_copy(
            table_hbm.at[idx_v.at[pl.ds(j * _CHUNK, _CHUNK)]], bufs[b], sin[b])

    def in_wait(b):
        pltpu.make_async_copy(
            table_hbm.at[pl.ds(0, _CHUNK)], bufs[b], sin[b]).wait()

    def out_start(j, b):
        pltpu.async_copy(
            bufs[b], out_hbm.at[pl.ds(base + j * _CHUNK, _CHUNK)], sout[b])

    def out_wait(b):
        pltpu.make_async_copy(
            bufs[b], out_hbm.at[pl.ds(base, _CHUNK)], sout[b]).wait()

    in_start(0, 0)
    in_start(1, 1)

    @pl.loop(0, nsteps - 2, step=2)
    def _(i):
        for k in range(2):
            j = i + k
            in_wait(k)
            out_start(j, k)
            out_wait(k)
            in_start(j + 2, k)

    for k in range(2):
        j = nsteps - 2 + k
        in_wait(k)
        out_start(j, k)
        out_wait(k)


def kernel(x, table):
    n = x.shape[0]
    b_per_w = n // _NW
    mesh = plsc.VectorSubcoreMesh(
        core_axis_name="c", subcore_axis_name="s",
        num_cores=_NC, num_subcores=_NS,
    )
    f = pl.kernel(
        _body,
        out_type=jax.ShapeDtypeStruct((n, _DIM), jnp.float32),
        mesh=mesh,
        scratch_types=[
            pltpu.VMEM((b_per_w,), jnp.int32),
            pltpu.VMEM((_CHUNK, _DIM), jnp.float32),
            pltpu.VMEM((_CHUNK, _DIM), jnp.float32),
            pltpu.SemaphoreType.DMA,
            pltpu.SemaphoreType.DMA,
            pltpu.SemaphoreType.DMA,
            pltpu.SemaphoreType.DMA,
        ],
    )
    return f(x.astype(jnp.int32), table)
